# Initial kernel scaffold; baseline (speedup 1.0000x reference)
import jax
import jax.numpy as jnp
from jax.experimental import pallas as pl


def kernel(features, edge_index, W1, b1, W2, b2):
    raise NotImplementedError("write your pallas kernel here")


# speedup vs baseline: 7.2229x; 7.2229x over previous
import functools

import jax
import jax.numpy as jnp
from jax import lax
from jax.experimental import pallas as pl
from jax.experimental.pallas import tpu as pltpu
from jax.experimental.pallas import tpu_sc as plsc

N = 100000
E = 1600000
FIN = 1433
FHID = 16
FOUT = 7

NC = 2
NS = 16
NW = NC * NS
EP = 1638400
CHUNK = 2048
EPW = EP // NW
EPT = EP // NS
TRASH = N

NPD = 100352
DSL = NPD // NS

NH = N // NC
NSP = 50176
ASL = NSP // NS

EB = 2000
MMB = 2000


def _zero_rows(ref, nrows, width):
  z = jnp.zeros((width,), jnp.float32)
  @pl.loop(0, nrows)
  def _(i):
    ref[i] = z


def _zero_flat(ref, n):
  z = jnp.zeros((16,), jnp.float32)
  @pl.loop(0, n // 16)
  def _(i):
    ref[pl.ds(i * 16, 16)] = z


def _deg_body(src2d, dst2d, dego_out, degi_out, dego_sh, degi_sh,
              sidx, didx, ones_v, zbuf):
  c = lax.axis_index("c")
  s = lax.axis_index("s")

  _zero_flat(zbuf, DSL)
  ones = jnp.ones((16,), jnp.float32)
  @pl.loop(0, 8)
  def _(i):
    ones_v[pl.ds(i * 16, 16)] = ones

  pltpu.sync_copy(zbuf, dego_sh.at[pl.ds(s * DSL, DSL)])
  pltpu.sync_copy(zbuf, degi_sh.at[pl.ds(s * DSL, DSL)])
  plsc.subcore_barrier()

  rbase = (c * NS + s) * (EPW // 128)
  @pl.loop(0, EPW // CHUNK)
  def _(k):
    base = rbase + k * 16
    pltpu.sync_copy(src2d.at[pl.ds(base, 16)], sidx)
    pltpu.sync_copy(dst2d.at[pl.ds(base, 16)], didx)
    @pl.loop(0, 16)
    def _(j):
      pltpu.sync_copy(ones_v, dego_sh.at[sidx.at[j]], add=True)
      pltpu.sync_copy(ones_v, degi_sh.at[didx.at[j]], add=True)

  plsc.subcore_barrier()
  pltpu.sync_copy(dego_sh.at[pl.ds(s * DSL, DSL)],
                  dego_out.at[c, pl.ds(s * DSL, DSL)])
  pltpu.sync_copy(degi_sh.at[pl.ds(s * DSL, DSL)],
                  degi_out.at[c, pl.ds(s * DSL, DSL)])


def _agg_body(h_hbm, srcg, dst2d, agg_out, agg_sh, sidx, didx, rows, zbuf):
  c = lax.axis_index("c")
  s = lax.axis_index("s")

  _zero_rows(zbuf, ASL // 8, FHID)
  @pl.loop(0, 8)
  def _(k):
    pltpu.sync_copy(zbuf, agg_sh.at[pl.ds(s * ASL + k * (ASL // 8), ASL // 8)])
  plsc.subcore_barrier()

  base = c * NH
  trash = NH + 8 + jnp.arange(16, dtype=jnp.int32) * 8

  @pl.loop(0, EPT // CHUNK)
  def _(k):
    pltpu.sync_copy(srcg.at[pl.ds(s * EPT + k * CHUNK, CHUNK)], sidx)
    pltpu.sync_copy(h_hbm.at[sidx], rows)
    pltpu.sync_copy(dst2d.at[pl.ds(s * (EPT // 128) + k * 16, 16)], didx)

    @pl.loop(0, 16)
    def _(j):
      @pl.loop(0, 8)
      def _(g):
        v = didx[j, pl.ds(g * 16, 16)] - base
        keep = (v >= 0) & (v < NH)
        didx[j, pl.ds(g * 16, 16)] = jnp.where(keep, v, trash)

    @pl.loop(0, 16)
    def _(j):
      pltpu.sync_copy(rows.at[pl.ds(j * 128, 128)],
                      agg_sh.at[didx.at[j]], add=True)

  plsc.subcore_barrier()
  pltpu.sync_copy(agg_sh.at[pl.ds(s * ASL, ASL)],
                  agg_out.at[c, pl.ds(s * ASL, ASL)])


@functools.cache
def _sc_kernels():
  mesh = plsc.VectorSubcoreMesh(core_axis_name="c", subcore_axis_name="s",
                                num_cores=NC, num_subcores=NS)
  deg = pl.kernel(
      _deg_body,
      out_type=(jax.ShapeDtypeStruct((NC, NPD), jnp.float32),
                jax.ShapeDtypeStruct((NC, NPD), jnp.float32)),
      mesh=mesh,
      scratch_types=[
          pltpu.VMEM_SHARED((NPD,), jnp.float32),
          pltpu.VMEM_SHARED((NPD,), jnp.float32),
          pltpu.VMEM((16, 128), jnp.int32),
          pltpu.VMEM((16, 128), jnp.int32),
          pltpu.VMEM((128,), jnp.float32),
          pltpu.VMEM((DSL,), jnp.float32),
      ],
  )
  agg = pl.kernel(
      _agg_body,
      out_type=jax.ShapeDtypeStruct((NC, NSP, FHID), jnp.float32),
      mesh=mesh,
      compiler_params=pltpu.CompilerParams(use_tc_tiling_on_sc=False),
      scratch_types=[
          pltpu.VMEM_SHARED((NSP, FHID), jnp.float32),
          pltpu.VMEM((CHUNK,), jnp.int32),
          pltpu.VMEM((16, 128), jnp.int32),
          pltpu.VMEM((CHUNK, FHID), jnp.float32),
          pltpu.VMEM((ASL // 8, FHID), jnp.float32),
      ],
  )
  return deg, agg


def _mm_body(x_ref, w_ref, o_ref):
  o_ref[...] = jnp.dot(x_ref[...], w_ref[...],
                       preferred_element_type=jnp.float32)


_mm_kernel = pl.pallas_call(
    _mm_body,
    grid=(N // MMB,),
    in_specs=[
        pl.BlockSpec((MMB, FIN), lambda i: (i, 0)),
        pl.BlockSpec((FIN, FHID), lambda i: (0, 0)),
    ],
    out_specs=pl.BlockSpec((MMB, FHID), lambda i: (i, 0)),
    out_shape=jax.ShapeDtypeStruct((N, FHID), jnp.float32),
)


def _norm(degp_ref):
  deg = degp_ref[:, 0] + degp_ref[:, 1]
  return lax.rsqrt(jnp.maximum(deg, 1.0))


def _e1_body(xw_ref, dego_ref, o_ref):
  o_ref[...] = xw_ref[...] * _norm(dego_ref)[:, None]


_e1_kernel = pl.pallas_call(
    _e1_body,
    grid=(N // EB,),
    in_specs=[
        pl.BlockSpec((EB, FHID), lambda i: (i, 0)),
        pl.BlockSpec((EB, NC), lambda i: (i, 0)),
    ],
    out_specs=pl.BlockSpec((EB, FHID), lambda i: (i, 0)),
    out_shape=jax.ShapeDtypeStruct((N, FHID), jnp.float32),
)


def _e2_body(agg_ref, dego_ref, degi_ref, b1_ref, o_ref):
  out1 = jax.nn.relu(agg_ref[...] * _norm(degi_ref)[:, None] + b1_ref[...])
  o_ref[...] = out1 * _norm(dego_ref)[:, None]


_e2_kernel = pl.pallas_call(
    _e2_body,
    grid=(N // EB,),
    in_specs=[
        pl.BlockSpec((EB, FHID), lambda i: (i, 0)),
        pl.BlockSpec((EB, NC), lambda i: (i, 0)),
        pl.BlockSpec((EB, NC), lambda i: (i, 0)),
        pl.BlockSpec((1, FHID), lambda i: (0, 0)),
    ],
    out_specs=pl.BlockSpec((EB, FHID), lambda i: (i, 0)),
    out_shape=jax.ShapeDtypeStruct((N, FHID), jnp.float32),
)


def _e3_body(agg_ref, degi_ref, w2_ref, b2_ref, o_ref):
  agg = agg_ref[...] * _norm(degi_ref)[:, None]
  o_ref[...] = jnp.dot(agg, w2_ref[...],
                       preferred_element_type=jnp.float32) + b2_ref[...]


_e3_kernel = pl.pallas_call(
    _e3_body,
    grid=(N // EB,),
    in_specs=[
        pl.BlockSpec((EB, FHID), lambda i: (i, 0)),
        pl.BlockSpec((EB, NC), lambda i: (i, 0)),
        pl.BlockSpec((FHID, FOUT), lambda i: (0, 0)),
        pl.BlockSpec((1, FOUT), lambda i: (0, 0)),
    ],
    out_specs=pl.BlockSpec((EB, FOUT), lambda i: (i, 0)),
    out_shape=jax.ShapeDtypeStruct((N, FOUT), jnp.float32),
)


def _assemble(aggp):
  return jnp.concatenate([aggp[0, :NH], aggp[1, :NH]], axis=0)


@jax.jit
def _run(features, edge_index, W1, b1, W2, b2):
  src = edge_index[0]
  dst = edge_index[1]
  pad = EP - E
  srcg = jnp.concatenate([src, jnp.zeros((pad,), jnp.int32)])
  trash = jnp.full((pad,), TRASH, jnp.int32)
  src2d = jnp.concatenate([src, trash]).reshape(EP // 128, 128)
  dst2d = jnp.concatenate([dst, trash]).reshape(EP // 128, 128)

  deg_kernel, agg_kernel = _sc_kernels()
  dego_p, degi_p = deg_kernel(src2d, dst2d)
  xw = _mm_kernel(features, W1)
  dego_t = dego_p.T
  degi_t = degi_p.T

  h1 = _e1_kernel(xw, dego_t)
  agg1 = _assemble(agg_kernel(h1, srcg, dst2d))
  g = _e2_kernel(agg1, dego_t, degi_t, b1.reshape(1, FHID))
  agg2 = _assemble(agg_kernel(g, srcg, dst2d))
  return _e3_kernel(agg2, degi_t, W2, b2.reshape(1, FOUT))


def kernel(features, edge_index, W1, b1, W2, b2):
  return _run(features, edge_index, W1, b1, W2, b2)


# speedup vs baseline: 7.6045x; 1.0528x over previous
import functools

import jax
import jax.numpy as jnp
from jax import lax
from jax.experimental import pallas as pl
from jax.experimental.pallas import tpu as pltpu
from jax.experimental.pallas import tpu_sc as plsc

N = 100000
E = 1600000
FIN = 1433
FHID = 16
FOUT = 7

NC = 2
NS = 16
NW = NC * NS
EP = 1638400
CHUNK = 2048
EPW = EP // NW
EPT = EP // NS
TRASH = N

NPD = 100352
DSL = NPD // NS

NH = N // NC
NSP = 50176
ASL = NSP // NS

EB = 2000
MMB = 2000


def _zero_rows(ref, nrows, width):
  z = jnp.zeros((width,), jnp.float32)
  @pl.loop(0, nrows)
  def _(i):
    ref[i] = z


def _zero_flat(ref, n):
  z = jnp.zeros((16,), jnp.float32)
  @pl.loop(0, n // 16)
  def _(i):
    ref[pl.ds(i * 16, 16)] = z


def _deg_body(src2d, dst2d, dego_out, degi_out, dego_sh, degi_sh,
              sidx, didx, ones_v, zbuf):
  c = lax.axis_index("c")
  s = lax.axis_index("s")

  _zero_flat(zbuf, DSL)
  ones = jnp.ones((16,), jnp.float32)
  @pl.loop(0, 8)
  def _(i):
    ones_v[pl.ds(i * 16, 16)] = ones

  pltpu.sync_copy(zbuf, dego_sh.at[pl.ds(s * DSL, DSL)])
  pltpu.sync_copy(zbuf, degi_sh.at[pl.ds(s * DSL, DSL)])
  plsc.subcore_barrier()

  rbase = (c * NS + s) * (EPW // 128)
  @pl.loop(0, EPW // CHUNK)
  def _(k):
    base = rbase + k * 16
    pltpu.sync_copy(src2d.at[pl.ds(base, 16)], sidx)
    pltpu.sync_copy(dst2d.at[pl.ds(base, 16)], didx)
    @pl.loop(0, 16)
    def _(j):
      pltpu.sync_copy(ones_v, dego_sh.at[sidx.at[j]], add=True)
      pltpu.sync_copy(ones_v, degi_sh.at[didx.at[j]], add=True)

  plsc.subcore_barrier()
  pltpu.sync_copy(dego_sh.at[pl.ds(s * DSL, DSL)],
                  dego_out.at[c, pl.ds(s * DSL, DSL)])
  pltpu.sync_copy(degi_sh.at[pl.ds(s * DSL, DSL)],
                  degi_out.at[c, pl.ds(s * DSL, DSL)])


def _agg_body(h_hbm, srcg, dst2d, agg_out, agg_sh, sidx, didx, rows, zbuf):
  c = lax.axis_index("c")
  s = lax.axis_index("s")

  _zero_rows(zbuf, ASL // 8, FHID)
  @pl.loop(0, 8)
  def _(k):
    pltpu.sync_copy(zbuf, agg_sh.at[pl.ds(s * ASL + k * (ASL // 8), ASL // 8)])
  plsc.subcore_barrier()

  base = c * NH
  trash = NH + 8 + jnp.arange(16, dtype=jnp.int32) * 8

  @pl.loop(0, EPT // CHUNK)
  def _(k):
    pltpu.sync_copy(srcg.at[pl.ds(s * EPT + k * CHUNK, CHUNK)], sidx)
    pltpu.sync_copy(h_hbm.at[sidx], rows)
    pltpu.sync_copy(dst2d.at[pl.ds(s * (EPT // 128) + k * 16, 16)], didx)

    @pl.loop(0, 16)
    def _(j):
      @pl.loop(0, 8)
      def _(g):
        v = didx[j, pl.ds(g * 16, 16)] - base
        keep = (v >= 0) & (v < NH)
        didx[j, pl.ds(g * 16, 16)] = jnp.where(keep, v, trash)

    @pl.loop(0, 16)
    def _(j):
      pltpu.sync_copy(rows.at[pl.ds(j * 128, 128)],
                      agg_sh.at[didx.at[j]], add=True)

  plsc.subcore_barrier()
  pltpu.sync_copy(agg_sh.at[pl.ds(s * ASL, ASL)],
                  agg_out.at[c, pl.ds(s * ASL, ASL)])


@functools.cache
def _sc_kernels():
  mesh = plsc.VectorSubcoreMesh(core_axis_name="c", subcore_axis_name="s",
                                num_cores=NC, num_subcores=NS)
  deg = pl.kernel(
      _deg_body,
      out_type=(jax.ShapeDtypeStruct((NC, NPD), jnp.float32),
                jax.ShapeDtypeStruct((NC, NPD), jnp.float32)),
      mesh=mesh,
      scratch_types=[
          pltpu.VMEM_SHARED((NPD,), jnp.float32),
          pltpu.VMEM_SHARED((NPD,), jnp.float32),
          pltpu.VMEM((16, 128), jnp.int32),
          pltpu.VMEM((16, 128), jnp.int32),
          pltpu.VMEM((128,), jnp.float32),
          pltpu.VMEM((DSL,), jnp.float32),
      ],
  )
  agg = pl.kernel(
      _agg_body,
      out_type=jax.ShapeDtypeStruct((NC, NSP, FHID), jnp.float32),
      mesh=mesh,
      compiler_params=pltpu.CompilerParams(use_tc_tiling_on_sc=False),
      scratch_types=[
          pltpu.VMEM_SHARED((NSP, FHID), jnp.float32),
          pltpu.VMEM((CHUNK,), jnp.int32),
          pltpu.VMEM((16, 128), jnp.int32),
          pltpu.VMEM((CHUNK, FHID), jnp.float32),
          pltpu.VMEM((ASL // 8, FHID), jnp.float32),
      ],
  )
  return deg, agg


def _mm_body(x_ref, w_ref, o_ref):
  o_ref[...] = jnp.dot(x_ref[...].astype(jnp.bfloat16),
                       w_ref[...].astype(jnp.bfloat16),
                       preferred_element_type=jnp.float32)


_mm_kernel = pl.pallas_call(
    _mm_body,
    grid=(N // MMB,),
    in_specs=[
        pl.BlockSpec((MMB, FIN), lambda i: (i, 0)),
        pl.BlockSpec((FIN, FHID), lambda i: (0, 0)),
    ],
    out_specs=pl.BlockSpec((MMB, FHID), lambda i: (i, 0)),
    out_shape=jax.ShapeDtypeStruct((N, FHID), jnp.float32),
)


def _norm(degp_ref):
  deg = degp_ref[:, 0] + degp_ref[:, 1]
  return lax.rsqrt(jnp.maximum(deg, 1.0))


def _e1_body(xw_ref, dego_ref, o_ref):
  o_ref[...] = xw_ref[...] * _norm(dego_ref)[:, None]


_e1_kernel = pl.pallas_call(
    _e1_body,
    grid=(N // EB,),
    in_specs=[
        pl.BlockSpec((EB, FHID), lambda i: (i, 0)),
        pl.BlockSpec((EB, NC), lambda i: (i, 0)),
    ],
    out_specs=pl.BlockSpec((EB, FHID), lambda i: (i, 0)),
    out_shape=jax.ShapeDtypeStruct((N, FHID), jnp.float32),
)


_PART = lambda i: (i // (NH // EB), i % (NH // EB), 0)


def _e2_body(agg_ref, dego_ref, degi_ref, b1_ref, o_ref):
  out1 = jax.nn.relu(agg_ref[0] * _norm(degi_ref)[:, None] + b1_ref[...])
  o_ref[...] = out1 * _norm(dego_ref)[:, None]


_e2_kernel = pl.pallas_call(
    _e2_body,
    grid=(N // EB,),
    in_specs=[
        pl.BlockSpec((1, EB, FHID), _PART),
        pl.BlockSpec((EB, NC), lambda i: (i, 0)),
        pl.BlockSpec((EB, NC), lambda i: (i, 0)),
        pl.BlockSpec((1, FHID), lambda i: (0, 0)),
    ],
    out_specs=pl.BlockSpec((EB, FHID), lambda i: (i, 0)),
    out_shape=jax.ShapeDtypeStruct((N, FHID), jnp.float32),
)


def _e3_body(agg_ref, degi_ref, w2_ref, b2_ref, o_ref):
  agg = agg_ref[0] * _norm(degi_ref)[:, None]
  o_ref[...] = jnp.dot(agg, w2_ref[...],
                       preferred_element_type=jnp.float32) + b2_ref[...]


_e3_kernel = pl.pallas_call(
    _e3_body,
    grid=(N // EB,),
    in_specs=[
        pl.BlockSpec((1, EB, FHID), _PART),
        pl.BlockSpec((EB, NC), lambda i: (i, 0)),
        pl.BlockSpec((FHID, FOUT), lambda i: (0, 0)),
        pl.BlockSpec((1, FOUT), lambda i: (0, 0)),
    ],
    out_specs=pl.BlockSpec((EB, FOUT), lambda i: (i, 0)),
    out_shape=jax.ShapeDtypeStruct((N, FOUT), jnp.float32),
)


@jax.jit
def _run(features, edge_index, W1, b1, W2, b2):
  src = edge_index[0]
  dst = edge_index[1]
  pad = EP - E
  srcg = jnp.concatenate([src, jnp.zeros((pad,), jnp.int32)])
  trash = jnp.full((pad,), TRASH, jnp.int32)
  src2d = jnp.concatenate([src, trash]).reshape(EP // 128, 128)
  dst2d = jnp.concatenate([dst, trash]).reshape(EP // 128, 128)

  deg_kernel, agg_kernel = _sc_kernels()
  dego_p, degi_p = deg_kernel(src2d, dst2d)
  xw = _mm_kernel(features, W1)
  dego_t = dego_p.T
  degi_t = degi_p.T

  h1 = _e1_kernel(xw, dego_t)
  agg1 = agg_kernel(h1, srcg, dst2d)
  g = _e2_kernel(agg1, dego_t, degi_t, b1.reshape(1, FHID))
  agg2 = agg_kernel(g, srcg, dst2d)
  return _e3_kernel(agg2, degi_t, W2, b2.reshape(1, FOUT))


def kernel(features, edge_index, W1, b1, W2, b2):
  return _run(features, edge_index, W1, b1, W2, b2)


# speedup vs baseline: 10.8563x; 1.4276x over previous
import functools

import jax
import jax.numpy as jnp
from jax import lax
from jax.experimental import pallas as pl
from jax.experimental.pallas import tpu as pltpu
from jax.experimental.pallas import tpu_sc as plsc

N = 100000
E = 1600000
FIN = 1433
FHID = 16
FOUT = 7

NC = 2
NS = 16
NW = NC * NS
EP = 1638400
CHUNK = 2048
EPW = EP // NW
EPT = EP // NS
TRASH = N

NPD = 100352
DSL = NPD // NS

NH = N // NC
NSP = 50176
ASL = NSP // NS

EB = 2000
MMB = 2000


def _zero_rows(ref, nrows, width):
  z = jnp.zeros((width,), jnp.float32)
  @pl.loop(0, nrows)
  def _(i):
    ref[i] = z


def _zero_flat(ref, n):
  z = jnp.zeros((16,), jnp.float32)
  @pl.loop(0, n // 16)
  def _(i):
    ref[pl.ds(i * 16, 16)] = z


def _deg_body(src2d, dst2d, dego_out, degi_out, dego_sh, degi_sh,
              sidx, didx, ones_v, zbuf):
  c = lax.axis_index("c")
  s = lax.axis_index("s")

  _zero_flat(zbuf, DSL)
  ones = jnp.ones((16,), jnp.float32)
  @pl.loop(0, 8)
  def _(i):
    ones_v[pl.ds(i * 16, 16)] = ones

  pltpu.sync_copy(zbuf, dego_sh.at[pl.ds(s * DSL, DSL)])
  pltpu.sync_copy(zbuf, degi_sh.at[pl.ds(s * DSL, DSL)])
  plsc.subcore_barrier()

  rbase = (c * NS + s) * (EPW // 128)
  @pl.loop(0, EPW // CHUNK)
  def _(k):
    base = rbase + k * 16
    pltpu.sync_copy(src2d.at[pl.ds(base, 16)], sidx)
    pltpu.sync_copy(dst2d.at[pl.ds(base, 16)], didx)
    @pl.loop(0, 16)
    def _(j):
      pltpu.sync_copy(ones_v, dego_sh.at[sidx.at[j]], add=True)
      pltpu.sync_copy(ones_v, degi_sh.at[didx.at[j]], add=True)

  plsc.subcore_barrier()
  pltpu.sync_copy(dego_sh.at[pl.ds(s * DSL, DSL)],
                  dego_out.at[c, pl.ds(s * DSL, DSL)])
  pltpu.sync_copy(degi_sh.at[pl.ds(s * DSL, DSL)],
                  degi_out.at[c, pl.ds(s * DSL, DSL)])


CCAP = 2304
TRASHROW = NH + 8


def _agg_body(h_hbm, srcg, dstg, agg_out, agg_sh, sidx, didx, csrc, cdst,
              didx2, grows, zbuf, cnt_ref, gsem, ssem):
  c = lax.axis_index("c")
  s = lax.axis_index("s")

  _zero_rows(zbuf, ASL // 8, FHID)
  @pl.loop(0, 8)
  def _(k):
    pltpu.sync_copy(zbuf, agg_sh.at[pl.ds(s * ASL + k * (ASL // 8), ASL // 8)])
  plsc.subcore_barrier()

  base = c * NH
  cnt_ref[0] = 0

  def flush_full():
    m = cnt_ref[0] // 128
    @pl.loop(0, m)
    def _(g):
      @pl.loop(0, 8)
      def _(i):
        didx2[g, pl.ds(i * 16, 16)] = cdst[pl.ds(g * 128 + i * 16, 16)]
      pltpu.async_copy(h_hbm.at[csrc.at[pl.ds(g * 128, 128)]],
                       grows.at[g], gsem)
    @pl.loop(0, m)
    def _(g):
      pltpu.make_async_copy(h_hbm.at[csrc.at[pl.ds(g * 128, 128)]],
                            grows.at[g], gsem).wait()
    @pl.loop(0, m)
    def _(g):
      pltpu.async_copy(grows.at[g], agg_sh.at[didx2.at[g]], ssem, add=True)
    @pl.loop(0, m)
    def _(g):
      pltpu.make_async_copy(grows.at[g], agg_sh.at[didx2.at[g]], ssem).wait()
    mb = m * 128
    @pl.loop(0, 8)
    def _(i):
      csrc[pl.ds(i * 16, 16)] = csrc[pl.ds(mb + i * 16, 16)]
      cdst[pl.ds(i * 16, 16)] = cdst[pl.ds(mb + i * 16, 16)]
    cnt_ref[0] = cnt_ref[0] - mb

  @pl.loop(0, EPT // CHUNK)
  def _(k):
    pltpu.sync_copy(srcg.at[pl.ds(s * EPT + k * CHUNK, CHUNK)], sidx)
    pltpu.sync_copy(dstg.at[pl.ds(s * EPT + k * CHUNK, CHUNK)], didx)
    @pl.loop(0, CHUNK // 16)
    def _(g):
      dv = didx[pl.ds(g * 16, 16)] - base
      keep = (dv >= 0) & (dv < NH)
      n = cnt_ref[0]
      plsc.store_compressed(cdst.at[pl.ds(n, 16)], dv, mask=keep)
      plsc.store_compressed(csrc.at[pl.ds(n, 16)], sidx[pl.ds(g * 16, 16)],
                            mask=keep)
      cnt_ref[0] = n + jnp.sum(keep.astype(jnp.int32))
    flush_full()

  r = cnt_ref[0]
  lane = jnp.arange(16, dtype=jnp.int32)
  @pl.loop(0, 8)
  def _(i):
    valid = (i * 16 + lane) < r
    cdst[pl.ds(i * 16, 16)] = jnp.where(
        valid, cdst[pl.ds(i * 16, 16)],
        jnp.full((16,), TRASHROW, jnp.int32))
    csrc[pl.ds(i * 16, 16)] = jnp.where(
        valid, csrc[pl.ds(i * 16, 16)], jnp.zeros((16,), jnp.int32))
  cnt_ref[0] = 128
  flush_full()

  plsc.subcore_barrier()
  pltpu.sync_copy(agg_sh.at[pl.ds(s * ASL, ASL)],
                  agg_out.at[c, pl.ds(s * ASL, ASL)])


@functools.cache
def _sc_kernels():
  mesh = plsc.VectorSubcoreMesh(core_axis_name="c", subcore_axis_name="s",
                                num_cores=NC, num_subcores=NS)
  deg = pl.kernel(
      _deg_body,
      out_type=(jax.ShapeDtypeStruct((NC, NPD), jnp.float32),
                jax.ShapeDtypeStruct((NC, NPD), jnp.float32)),
      mesh=mesh,
      scratch_types=[
          pltpu.VMEM_SHARED((NPD,), jnp.float32),
          pltpu.VMEM_SHARED((NPD,), jnp.float32),
          pltpu.VMEM((16, 128), jnp.int32),
          pltpu.VMEM((16, 128), jnp.int32),
          pltpu.VMEM((128,), jnp.float32),
          pltpu.VMEM((DSL,), jnp.float32),
      ],
  )
  agg = pl.kernel(
      _agg_body,
      out_type=jax.ShapeDtypeStruct((NC, NSP, FHID), jnp.float32),
      mesh=mesh,
      compiler_params=pltpu.CompilerParams(use_tc_tiling_on_sc=False,
                                           needs_layout_passes=False),
      scratch_types=[
          pltpu.VMEM_SHARED((NSP, FHID), jnp.float32),
          pltpu.VMEM((CHUNK,), jnp.int32),
          pltpu.VMEM((CHUNK,), jnp.int32),
          pltpu.VMEM((CCAP,), jnp.int32),
          pltpu.VMEM((CCAP,), jnp.int32),
          pltpu.VMEM((17, 128), jnp.int32),
          pltpu.VMEM((17, 128, FHID), jnp.float32),
          pltpu.VMEM((ASL // 8, FHID), jnp.float32),
          pltpu.SMEM((1,), jnp.int32),
          pltpu.SemaphoreType.DMA,
          pltpu.SemaphoreType.DMA,
      ],
  )
  return deg, agg


def _mm_body(x_ref, w_ref, o_ref):
  o_ref[...] = jnp.dot(x_ref[...].astype(jnp.bfloat16),
                       w_ref[...].astype(jnp.bfloat16),
                       preferred_element_type=jnp.float32)


_mm_kernel = pl.pallas_call(
    _mm_body,
    grid=(N // MMB,),
    in_specs=[
        pl.BlockSpec((MMB, FIN), lambda i: (i, 0)),
        pl.BlockSpec((FIN, FHID), lambda i: (0, 0)),
    ],
    out_specs=pl.BlockSpec((MMB, FHID), lambda i: (i, 0)),
    out_shape=jax.ShapeDtypeStruct((N, FHID), jnp.float32),
)


def _norm(degp_ref):
  deg = degp_ref[:, 0] + degp_ref[:, 1]
  return lax.rsqrt(jnp.maximum(deg, 1.0))


def _e1_body(xw_ref, dego_ref, o_ref):
  o_ref[...] = xw_ref[...] * _norm(dego_ref)[:, None]


_e1_kernel = pl.pallas_call(
    _e1_body,
    grid=(N // EB,),
    in_specs=[
        pl.BlockSpec((EB, FHID), lambda i: (i, 0)),
        pl.BlockSpec((EB, NC), lambda i: (i, 0)),
    ],
    out_specs=pl.BlockSpec((EB, FHID), lambda i: (i, 0)),
    out_shape=jax.ShapeDtypeStruct((N, FHID), jnp.float32),
)


_PART = lambda i: (i // (NH // EB), i % (NH // EB), 0)


def _e2_body(agg_ref, dego_ref, degi_ref, b1_ref, o_ref):
  out1 = jax.nn.relu(agg_ref[0] * _norm(degi_ref)[:, None] + b1_ref[...])
  o_ref[...] = out1 * _norm(dego_ref)[:, None]


_e2_kernel = pl.pallas_call(
    _e2_body,
    grid=(N // EB,),
    in_specs=[
        pl.BlockSpec((1, EB, FHID), _PART),
        pl.BlockSpec((EB, NC), lambda i: (i, 0)),
        pl.BlockSpec((EB, NC), lambda i: (i, 0)),
        pl.BlockSpec((1, FHID), lambda i: (0, 0)),
    ],
    out_specs=pl.BlockSpec((EB, FHID), lambda i: (i, 0)),
    out_shape=jax.ShapeDtypeStruct((N, FHID), jnp.float32),
)


def _e3_body(agg_ref, degi_ref, w2_ref, b2_ref, o_ref):
  agg = agg_ref[0] * _norm(degi_ref)[:, None]
  o_ref[...] = jnp.dot(agg, w2_ref[...],
                       preferred_element_type=jnp.float32) + b2_ref[...]


_e3_kernel = pl.pallas_call(
    _e3_body,
    grid=(N // EB,),
    in_specs=[
        pl.BlockSpec((1, EB, FHID), _PART),
        pl.BlockSpec((EB, NC), lambda i: (i, 0)),
        pl.BlockSpec((FHID, FOUT), lambda i: (0, 0)),
        pl.BlockSpec((1, FOUT), lambda i: (0, 0)),
    ],
    out_specs=pl.BlockSpec((EB, FOUT), lambda i: (i, 0)),
    out_shape=jax.ShapeDtypeStruct((N, FOUT), jnp.float32),
)


@jax.jit
def _run(features, edge_index, W1, b1, W2, b2):
  src = edge_index[0]
  dst = edge_index[1]
  pad = EP - E
  srcg = jnp.concatenate([src, jnp.zeros((pad,), jnp.int32)])
  trash = jnp.full((pad,), TRASH, jnp.int32)
  dstg = jnp.concatenate([dst, trash])
  src2d = jnp.concatenate([src, trash]).reshape(EP // 128, 128)
  dst2d = dstg.reshape(EP // 128, 128)

  deg_kernel, agg_kernel = _sc_kernels()
  dego_p, degi_p = deg_kernel(src2d, dst2d)
  xw = _mm_kernel(features, W1)
  dego_t = dego_p.T
  degi_t = degi_p.T

  h1 = _e1_kernel(xw, dego_t)
  agg1 = agg_kernel(h1, srcg, dstg)
  g = _e2_kernel(agg1, dego_t, degi_t, b1.reshape(1, FHID))
  agg2 = agg_kernel(g, srcg, dstg)
  return _e3_kernel(agg2, degi_t, W2, b2.reshape(1, FOUT))


def kernel(features, edge_index, W1, b1, W2, b2):
  return _run(features, edge_index, W1, b1, W2, b2)


# speedup vs baseline: 10.9330x; 1.0071x over previous
import functools

import jax
import jax.numpy as jnp
from jax import lax
from jax.experimental import pallas as pl
from jax.experimental.pallas import tpu as pltpu
from jax.experimental.pallas import tpu_sc as plsc

N = 100000
E = 1600000
FIN = 1433
FHID = 16
FOUT = 7

NC = 2
NS = 16
NW = NC * NS
EP = 1638400
CHUNK = 2048
EPW = EP // NW
EPT = EP // NS
TRASH = N

NPD = 100352
DSL = NPD // NS

NH = N // NC
NSP = 50176
ASL = NSP // NS

EB = 10000
MMB = 2000


def _zero_rows(ref, nrows, width):
  z = jnp.zeros((width,), jnp.float32)
  @pl.loop(0, nrows)
  def _(i):
    ref[i] = z


def _zero_flat(ref, n):
  z = jnp.zeros((16,), jnp.float32)
  @pl.loop(0, n // 16)
  def _(i):
    ref[pl.ds(i * 16, 16)] = z


def _deg_body(src2d, dst2d, dego_out, degi_out, dego_sh, degi_sh,
              sidx, didx, ones_v, zbuf):
  c = lax.axis_index("c")
  s = lax.axis_index("s")

  _zero_flat(zbuf, DSL)
  ones = jnp.ones((16,), jnp.float32)
  @pl.loop(0, 8)
  def _(i):
    ones_v[pl.ds(i * 16, 16)] = ones

  pltpu.sync_copy(zbuf, dego_sh.at[pl.ds(s * DSL, DSL)])
  pltpu.sync_copy(zbuf, degi_sh.at[pl.ds(s * DSL, DSL)])
  plsc.subcore_barrier()

  rbase = (c * NS + s) * (EPW // 128)
  @pl.loop(0, EPW // CHUNK)
  def _(k):
    base = rbase + k * 16
    pltpu.sync_copy(src2d.at[pl.ds(base, 16)], sidx)
    pltpu.sync_copy(dst2d.at[pl.ds(base, 16)], didx)
    @pl.loop(0, 16)
    def _(j):
      pltpu.sync_copy(ones_v, dego_sh.at[sidx.at[j]], add=True)
      pltpu.sync_copy(ones_v, degi_sh.at[didx.at[j]], add=True)

  plsc.subcore_barrier()
  pltpu.sync_copy(dego_sh.at[pl.ds(s * DSL, DSL)],
                  dego_out.at[c, pl.ds(s * DSL, DSL)])
  pltpu.sync_copy(degi_sh.at[pl.ds(s * DSL, DSL)],
                  degi_out.at[c, pl.ds(s * DSL, DSL)])


CCAP = 2304
TRASHROW = NH + 8


def _agg_body(h_hbm, srcg, dstg, agg_out, agg_sh, sidx, didx, csrc, cdst,
              didx2, grows, zbuf, cnt_ref, gsem, ssem):
  c = lax.axis_index("c")
  s = lax.axis_index("s")

  _zero_rows(zbuf, ASL // 8, FHID)
  @pl.loop(0, 8)
  def _(k):
    pltpu.sync_copy(zbuf, agg_sh.at[pl.ds(s * ASL + k * (ASL // 8), ASL // 8)])
  plsc.subcore_barrier()

  base = c * NH
  cnt_ref[0] = 0

  def flush_full():
    m = cnt_ref[0] // 128
    @pl.loop(0, m)
    def _(g):
      @pl.loop(0, 8)
      def _(i):
        didx2[g, pl.ds(i * 16, 16)] = cdst[pl.ds(g * 128 + i * 16, 16)]
      pltpu.async_copy(h_hbm.at[csrc.at[pl.ds(g * 128, 128)]],
                       grows.at[g], gsem)
    @pl.loop(0, m)
    def _(g):
      pltpu.make_async_copy(h_hbm.at[csrc.at[pl.ds(g * 128, 128)]],
                            grows.at[g], gsem).wait()
    @pl.loop(0, m)
    def _(g):
      pltpu.async_copy(grows.at[g], agg_sh.at[didx2.at[g]], ssem, add=True)
    @pl.loop(0, m)
    def _(g):
      pltpu.make_async_copy(grows.at[g], agg_sh.at[didx2.at[g]], ssem).wait()
    mb = m * 128
    @pl.loop(0, 8)
    def _(i):
      csrc[pl.ds(i * 16, 16)] = csrc[pl.ds(mb + i * 16, 16)]
      cdst[pl.ds(i * 16, 16)] = cdst[pl.ds(mb + i * 16, 16)]
    cnt_ref[0] = cnt_ref[0] - mb

  @pl.loop(0, EPT // CHUNK)
  def _(k):
    pltpu.sync_copy(srcg.at[pl.ds(s * EPT + k * CHUNK, CHUNK)], sidx)
    pltpu.sync_copy(dstg.at[pl.ds(s * EPT + k * CHUNK, CHUNK)], didx)
    @pl.loop(0, CHUNK // 16)
    def _(g):
      dv = didx[pl.ds(g * 16, 16)] - base
      keep = (dv >= 0) & (dv < NH)
      n = cnt_ref[0]
      plsc.store_compressed(cdst.at[pl.ds(n, 16)], dv, mask=keep)
      plsc.store_compressed(csrc.at[pl.ds(n, 16)], sidx[pl.ds(g * 16, 16)],
                            mask=keep)
      cnt_ref[0] = n + jnp.sum(keep.astype(jnp.int32))
    flush_full()

  r = cnt_ref[0]
  lane = jnp.arange(16, dtype=jnp.int32)
  @pl.loop(0, 8)
  def _(i):
    valid = (i * 16 + lane) < r
    cdst[pl.ds(i * 16, 16)] = jnp.where(
        valid, cdst[pl.ds(i * 16, 16)],
        jnp.full((16,), TRASHROW, jnp.int32))
    csrc[pl.ds(i * 16, 16)] = jnp.where(
        valid, csrc[pl.ds(i * 16, 16)], jnp.zeros((16,), jnp.int32))
  cnt_ref[0] = 128
  flush_full()

  plsc.subcore_barrier()
  pltpu.sync_copy(agg_sh.at[pl.ds(s * ASL, ASL)],
                  agg_out.at[c, pl.ds(s * ASL, ASL)])


@functools.cache
def _sc_kernels():
  mesh = plsc.VectorSubcoreMesh(core_axis_name="c", subcore_axis_name="s",
                                num_cores=NC, num_subcores=NS)
  deg = pl.kernel(
      _deg_body,
      out_type=(jax.ShapeDtypeStruct((NC, NPD), jnp.float32),
                jax.ShapeDtypeStruct((NC, NPD), jnp.float32)),
      mesh=mesh,
      scratch_types=[
          pltpu.VMEM_SHARED((NPD,), jnp.float32),
          pltpu.VMEM_SHARED((NPD,), jnp.float32),
          pltpu.VMEM((16, 128), jnp.int32),
          pltpu.VMEM((16, 128), jnp.int32),
          pltpu.VMEM((128,), jnp.float32),
          pltpu.VMEM((DSL,), jnp.float32),
      ],
  )
  agg = pl.kernel(
      _agg_body,
      out_type=jax.ShapeDtypeStruct((NC, NSP, FHID), jnp.float32),
      mesh=mesh,
      compiler_params=pltpu.CompilerParams(use_tc_tiling_on_sc=False,
                                           needs_layout_passes=False),
      scratch_types=[
          pltpu.VMEM_SHARED((NSP, FHID), jnp.float32),
          pltpu.VMEM((CHUNK,), jnp.int32),
          pltpu.VMEM((CHUNK,), jnp.int32),
          pltpu.VMEM((CCAP,), jnp.int32),
          pltpu.VMEM((CCAP,), jnp.int32),
          pltpu.VMEM((17, 128), jnp.int32),
          pltpu.VMEM((17, 128, FHID), jnp.float32),
          pltpu.VMEM((ASL // 8, FHID), jnp.float32),
          pltpu.SMEM((1,), jnp.int32),
          pltpu.SemaphoreType.DMA,
          pltpu.SemaphoreType.DMA,
      ],
  )
  return deg, agg


def _mm_body(x_ref, w_ref, o_ref):
  o_ref[...] = jnp.dot(x_ref[...], w_ref[...],
                       preferred_element_type=jnp.float32)


_mm_kernel = pl.pallas_call(
    _mm_body,
    grid=(N // MMB,),
    in_specs=[
        pl.BlockSpec((MMB, FIN), lambda i: (i, 0)),
        pl.BlockSpec((FIN, FHID), lambda i: (0, 0)),
    ],
    out_specs=pl.BlockSpec((MMB, FHID), lambda i: (i, 0)),
    out_shape=jax.ShapeDtypeStruct((N, FHID), jnp.float32),
)


def _norm(degp_ref):
  deg = degp_ref[:, 0] + degp_ref[:, 1]
  return lax.rsqrt(jnp.maximum(deg, 1.0))


def _e1_body(xw_ref, dego_ref, o_ref):
  o_ref[...] = xw_ref[...] * _norm(dego_ref)[:, None]


_e1_kernel = pl.pallas_call(
    _e1_body,
    grid=(N // EB,),
    in_specs=[
        pl.BlockSpec((EB, FHID), lambda i: (i, 0)),
        pl.BlockSpec((EB, NC), lambda i: (i, 0)),
    ],
    out_specs=pl.BlockSpec((EB, FHID), lambda i: (i, 0)),
    out_shape=jax.ShapeDtypeStruct((N, FHID), jnp.float32),
)


_PART = lambda i: (i // (NH // EB), i % (NH // EB), 0)


def _e2_body(agg_ref, dego_ref, degi_ref, b1_ref, o_ref):
  out1 = jax.nn.relu(agg_ref[0] * _norm(degi_ref)[:, None] + b1_ref[...])
  o_ref[...] = out1 * _norm(dego_ref)[:, None]


_e2_kernel = pl.pallas_call(
    _e2_body,
    grid=(N // EB,),
    in_specs=[
        pl.BlockSpec((1, EB, FHID), _PART),
        pl.BlockSpec((EB, NC), lambda i: (i, 0)),
        pl.BlockSpec((EB, NC), lambda i: (i, 0)),
        pl.BlockSpec((1, FHID), lambda i: (0, 0)),
    ],
    out_specs=pl.BlockSpec((EB, FHID), lambda i: (i, 0)),
    out_shape=jax.ShapeDtypeStruct((N, FHID), jnp.float32),
)


def _e3_body(agg_ref, degi_ref, w2_ref, b2_ref, o_ref):
  agg = agg_ref[0] * _norm(degi_ref)[:, None]
  o_ref[...] = jnp.dot(agg, w2_ref[...],
                       preferred_element_type=jnp.float32) + b2_ref[...]


_e3_kernel = pl.pallas_call(
    _e3_body,
    grid=(N // EB,),
    in_specs=[
        pl.BlockSpec((1, EB, FHID), _PART),
        pl.BlockSpec((EB, NC), lambda i: (i, 0)),
        pl.BlockSpec((FHID, FOUT), lambda i: (0, 0)),
        pl.BlockSpec((1, FOUT), lambda i: (0, 0)),
    ],
    out_specs=pl.BlockSpec((EB, FOUT), lambda i: (i, 0)),
    out_shape=jax.ShapeDtypeStruct((N, FOUT), jnp.float32),
)


@jax.jit
def _run(features, edge_index, W1, b1, W2, b2):
  src = edge_index[0]
  dst = edge_index[1]
  pad = EP - E
  srcg = jnp.concatenate([src, jnp.zeros((pad,), jnp.int32)])
  trash = jnp.full((pad,), TRASH, jnp.int32)
  dstg = jnp.concatenate([dst, trash])
  src2d = jnp.concatenate([src, trash]).reshape(EP // 128, 128)
  dst2d = dstg.reshape(EP // 128, 128)

  deg_kernel, agg_kernel = _sc_kernels()
  dego_p, degi_p = deg_kernel(src2d, dst2d)
  xw = _mm_kernel(features.astype(jnp.bfloat16),
                  W1.astype(jnp.bfloat16))
  dego_t = dego_p.T
  degi_t = degi_p.T

  h1 = _e1_kernel(xw, dego_t)
  agg1 = agg_kernel(h1, srcg, dstg)
  g = _e2_kernel(agg1, dego_t, degi_t, b1.reshape(1, FHID))
  agg2 = agg_kernel(g, srcg, dstg)
  return _e3_kernel(agg2, degi_t, W2, b2.reshape(1, FOUT))


def kernel(features, edge_index, W1, b1, W2, b2):
  return _run(features, edge_index, W1, b1, W2, b2)


# speedup vs baseline: 16.0377x; 1.4669x over previous
import functools

import jax
import jax.numpy as jnp
from jax import lax
from jax.experimental import pallas as pl
from jax.experimental.pallas import tpu as pltpu
from jax.experimental.pallas import tpu_sc as plsc

N = 100000
E = 1600000
FIN = 1433
FHID = 16
FOUT = 7

NC = 2
NS = 16
NW = NC * NS
EP = 1638400
CHUNK = 2048
EPW = EP // NW
EPT = EP // NS
TRASH = N

NPD = 100352
DSL = NPD // NS

NH = N // NC
NSP = 50176
ASL = NSP // NS

EB = 10000
MMB = 2048


def _zero_rows(ref, nrows, width):
  z = jnp.zeros((width,), jnp.float32)
  @pl.loop(0, nrows)
  def _(i):
    ref[i] = z


def _zero_flat(ref, n):
  z = jnp.zeros((16,), jnp.float32)
  @pl.loop(0, n // 16)
  def _(i):
    ref[pl.ds(i * 16, 16)] = z


def _deg_body(src2d, dst2d, dego_out, degi_out, dego_sh, degi_sh,
              sidx, didx, ones_v, zbuf):
  c = lax.axis_index("c")
  s = lax.axis_index("s")

  _zero_flat(zbuf, DSL)
  ones = jnp.ones((16,), jnp.float32)
  @pl.loop(0, 8)
  def _(i):
    ones_v[pl.ds(i * 16, 16)] = ones

  pltpu.sync_copy(zbuf, dego_sh.at[pl.ds(s * DSL, DSL)])
  pltpu.sync_copy(zbuf, degi_sh.at[pl.ds(s * DSL, DSL)])
  plsc.subcore_barrier()

  rbase = (c * NS + s) * (EPW // 128)
  @pl.loop(0, EPW // CHUNK)
  def _(k):
    base = rbase + k * 16
    pltpu.sync_copy(src2d.at[pl.ds(base, 16)], sidx)
    pltpu.sync_copy(dst2d.at[pl.ds(base, 16)], didx)
    @pl.loop(0, 16)
    def _(j):
      pltpu.sync_copy(ones_v, dego_sh.at[sidx.at[j]], add=True)
      pltpu.sync_copy(ones_v, degi_sh.at[didx.at[j]], add=True)

  plsc.subcore_barrier()
  pltpu.sync_copy(dego_sh.at[pl.ds(s * DSL, DSL)],
                  dego_out.at[c, pl.ds(s * DSL, DSL)])
  pltpu.sync_copy(degi_sh.at[pl.ds(s * DSL, DSL)],
                  degi_out.at[c, pl.ds(s * DSL, DSL)])


CCAP = 2304
TRASHROW = NH + 8


def _agg_body(h_hbm, srcg, dstg, agg_out, agg_sh, sidx, didx, csrc, cdst,
              didx2, grows, zbuf, cnt_ref, gsem, ssem):
  c = lax.axis_index("c")
  s = lax.axis_index("s")

  _zero_rows(zbuf, ASL // 8, FHID)
  @pl.loop(0, 8)
  def _(k):
    pltpu.sync_copy(zbuf, agg_sh.at[pl.ds(s * ASL + k * (ASL // 8), ASL // 8)])
  plsc.subcore_barrier()

  base = c * NH
  cnt_ref[0] = 0

  def flush_full():
    m = cnt_ref[0] // 128
    @pl.loop(0, m)
    def _(g):
      @pl.loop(0, 8)
      def _(i):
        didx2[g, pl.ds(i * 16, 16)] = cdst[pl.ds(g * 128 + i * 16, 16)]
      pltpu.async_copy(h_hbm.at[csrc.at[pl.ds(g * 128, 128)]],
                       grows.at[g], gsem)
    @pl.loop(0, m)
    def _(g):
      pltpu.make_async_copy(h_hbm.at[csrc.at[pl.ds(g * 128, 128)]],
                            grows.at[g], gsem).wait()
    @pl.loop(0, m)
    def _(g):
      pltpu.async_copy(grows.at[g], agg_sh.at[didx2.at[g]], ssem, add=True)
    @pl.loop(0, m)
    def _(g):
      pltpu.make_async_copy(grows.at[g], agg_sh.at[didx2.at[g]], ssem).wait()
    mb = m * 128
    @pl.loop(0, 8)
    def _(i):
      csrc[pl.ds(i * 16, 16)] = csrc[pl.ds(mb + i * 16, 16)]
      cdst[pl.ds(i * 16, 16)] = cdst[pl.ds(mb + i * 16, 16)]
    cnt_ref[0] = cnt_ref[0] - mb

  @pl.loop(0, EPT // CHUNK)
  def _(k):
    pltpu.sync_copy(srcg.at[pl.ds(s * EPT + k * CHUNK, CHUNK)], sidx)
    pltpu.sync_copy(dstg.at[pl.ds(s * EPT + k * CHUNK, CHUNK)], didx)
    @pl.loop(0, CHUNK // 16)
    def _(g):
      dv = didx[pl.ds(g * 16, 16)] - base
      keep = (dv >= 0) & (dv < NH)
      n = cnt_ref[0]
      plsc.store_compressed(cdst.at[pl.ds(n, 16)], dv, mask=keep)
      plsc.store_compressed(csrc.at[pl.ds(n, 16)], sidx[pl.ds(g * 16, 16)],
                            mask=keep)
      cnt_ref[0] = n + jnp.sum(keep.astype(jnp.int32))
    flush_full()

  r = cnt_ref[0]
  lane = jnp.arange(16, dtype=jnp.int32)
  @pl.loop(0, 8)
  def _(i):
    valid = (i * 16 + lane) < r
    cdst[pl.ds(i * 16, 16)] = jnp.where(
        valid, cdst[pl.ds(i * 16, 16)],
        jnp.full((16,), TRASHROW, jnp.int32))
    csrc[pl.ds(i * 16, 16)] = jnp.where(
        valid, csrc[pl.ds(i * 16, 16)], jnp.zeros((16,), jnp.int32))
  cnt_ref[0] = 128
  flush_full()

  plsc.subcore_barrier()
  pltpu.sync_copy(agg_sh.at[pl.ds(s * ASL, ASL)],
                  agg_out.at[c, pl.ds(s * ASL, ASL)])


@functools.cache
def _sc_kernels():
  mesh = plsc.VectorSubcoreMesh(core_axis_name="c", subcore_axis_name="s",
                                num_cores=NC, num_subcores=NS)
  deg = pl.kernel(
      _deg_body,
      out_type=(jax.ShapeDtypeStruct((NC, NPD), jnp.float32),
                jax.ShapeDtypeStruct((NC, NPD), jnp.float32)),
      mesh=mesh,
      scratch_types=[
          pltpu.VMEM_SHARED((NPD,), jnp.float32),
          pltpu.VMEM_SHARED((NPD,), jnp.float32),
          pltpu.VMEM((16, 128), jnp.int32),
          pltpu.VMEM((16, 128), jnp.int32),
          pltpu.VMEM((128,), jnp.float32),
          pltpu.VMEM((DSL,), jnp.float32),
      ],
  )
  agg = pl.kernel(
      _agg_body,
      out_type=jax.ShapeDtypeStruct((NC, NSP, FHID), jnp.float32),
      mesh=mesh,
      compiler_params=pltpu.CompilerParams(use_tc_tiling_on_sc=False,
                                           needs_layout_passes=False),
      scratch_types=[
          pltpu.VMEM_SHARED((NSP, FHID), jnp.float32),
          pltpu.VMEM((CHUNK,), jnp.int32),
          pltpu.VMEM((CHUNK,), jnp.int32),
          pltpu.VMEM((CCAP,), jnp.int32),
          pltpu.VMEM((CCAP,), jnp.int32),
          pltpu.VMEM((17, 128), jnp.int32),
          pltpu.VMEM((17, 128, FHID), jnp.float32),
          pltpu.VMEM((ASL // 8, FHID), jnp.float32),
          pltpu.SMEM((1,), jnp.int32),
          pltpu.SemaphoreType.DMA,
          pltpu.SemaphoreType.DMA,
      ],
  )
  return deg, agg


def _mm_body(xt_ref, w_ref, o_ref):
  o_ref[...] = lax.dot_general(
      xt_ref[...].astype(jnp.bfloat16), w_ref[...].astype(jnp.bfloat16),
      dimension_numbers=(((0,), (0,)), ((), ())),
      preferred_element_type=jnp.float32)


_mm_kernel = pl.pallas_call(
    _mm_body,
    grid=((N + MMB - 1) // MMB,),
    in_specs=[
        pl.BlockSpec((FIN, MMB), lambda i: (0, i)),
        pl.BlockSpec((FIN, FHID), lambda i: (0, 0)),
    ],
    out_specs=pl.BlockSpec((MMB, FHID), lambda i: (i, 0)),
    out_shape=jax.ShapeDtypeStruct((N, FHID), jnp.float32),
)


def _norm(degp_ref):
  deg = degp_ref[:, 0] + degp_ref[:, 1]
  return lax.rsqrt(jnp.maximum(deg, 1.0))


def _e1_body(xw_ref, dego_ref, o_ref):
  o_ref[...] = xw_ref[...] * _norm(dego_ref)[:, None]


_e1_kernel = pl.pallas_call(
    _e1_body,
    grid=(N // EB,),
    in_specs=[
        pl.BlockSpec((EB, FHID), lambda i: (i, 0)),
        pl.BlockSpec((EB, NC), lambda i: (i, 0)),
    ],
    out_specs=pl.BlockSpec((EB, FHID), lambda i: (i, 0)),
    out_shape=jax.ShapeDtypeStruct((N, FHID), jnp.float32),
)


_PART = lambda i: (i // (NH // EB), i % (NH // EB), 0)


def _e2_body(agg_ref, dego_ref, degi_ref, b1_ref, o_ref):
  out1 = jax.nn.relu(agg_ref[0] * _norm(degi_ref)[:, None] + b1_ref[...])
  o_ref[...] = out1 * _norm(dego_ref)[:, None]


_e2_kernel = pl.pallas_call(
    _e2_body,
    grid=(N // EB,),
    in_specs=[
        pl.BlockSpec((1, EB, FHID), _PART),
        pl.BlockSpec((EB, NC), lambda i: (i, 0)),
        pl.BlockSpec((EB, NC), lambda i: (i, 0)),
        pl.BlockSpec((1, FHID), lambda i: (0, 0)),
    ],
    out_specs=pl.BlockSpec((EB, FHID), lambda i: (i, 0)),
    out_shape=jax.ShapeDtypeStruct((N, FHID), jnp.float32),
)


def _e3_body(agg_ref, degi_ref, w2_ref, b2_ref, o_ref):
  agg = agg_ref[0] * _norm(degi_ref)[:, None]
  o_ref[...] = jnp.dot(agg, w2_ref[...],
                       preferred_element_type=jnp.float32) + b2_ref[...]


_e3_kernel = pl.pallas_call(
    _e3_body,
    grid=(N // EB,),
    in_specs=[
        pl.BlockSpec((1, EB, FHID), _PART),
        pl.BlockSpec((EB, NC), lambda i: (i, 0)),
        pl.BlockSpec((FHID, FOUT), lambda i: (0, 0)),
        pl.BlockSpec((1, FOUT), lambda i: (0, 0)),
    ],
    out_specs=pl.BlockSpec((EB, FOUT), lambda i: (i, 0)),
    out_shape=jax.ShapeDtypeStruct((N, FOUT), jnp.float32),
)


@jax.jit
def _run(features, edge_index, W1, b1, W2, b2):
  src = edge_index[0]
  dst = edge_index[1]
  pad = EP - E
  srcg = jnp.concatenate([src, jnp.zeros((pad,), jnp.int32)])
  trash = jnp.full((pad,), TRASH, jnp.int32)
  dstg = jnp.concatenate([dst, trash])
  src2d = jnp.concatenate([src, trash]).reshape(EP // 128, 128)
  dst2d = dstg.reshape(EP // 128, 128)

  deg_kernel, agg_kernel = _sc_kernels()
  dego_p, degi_p = deg_kernel(src2d, dst2d)
  xw = _mm_kernel(features.T, W1)
  dego_t = dego_p.T
  degi_t = degi_p.T

  h1 = _e1_kernel(xw, dego_t)
  agg1 = agg_kernel(h1, srcg, dstg)
  g = _e2_kernel(agg1, dego_t, degi_t, b1.reshape(1, FHID))
  agg2 = agg_kernel(g, srcg, dstg)
  return _e3_kernel(agg2, degi_t, W2, b2.reshape(1, FOUT))


def kernel(features, edge_index, W1, b1, W2, b2):
  return _run(features, edge_index, W1, b1, W2, b2)
